# Initial kernel scaffold; baseline (speedup 1.0000x reference)
import jax
import jax.numpy as jnp
from jax.experimental import pallas as pl


def kernel(x, edge_index, W, b, in_emb, out_emb):
    raise NotImplementedError("write your pallas kernel here")


# speedup vs baseline: 1.7829x; 1.7829x over previous
import functools

import jax
import jax.numpy as jnp
from jax import lax
from jax.experimental import pallas as pl
from jax.experimental.pallas import tpu as pltpu

N_NODES = 100000
EMB_DIM = 128
MAX_DEG = 256
BLK = 2000


def _tc_body(x_ref, wt_ref, b_ref, ind_ref, outd_ref, ie_ref, oe_ref, o_ref):
    h = jnp.dot(x_ref[...], wt_ref[...], preferred_element_type=jnp.float32)
    h = h + b_ref[...]
    iota = lax.broadcasted_iota(jnp.int32, (BLK, MAX_DEG), 1)
    oh_in = (ind_ref[...] == iota).astype(jnp.float32)
    oh_out = (outd_ref[...] == iota).astype(jnp.float32)
    h = h + jnp.dot(oh_in, ie_ref[...], preferred_element_type=jnp.float32)
    h = h + jnp.dot(oh_out, oe_ref[...], preferred_element_type=jnp.float32)
    o_ref[...] = h


@jax.jit
def _tc_pass(x, Wt, b2, in_deg, out_deg, in_emb, out_emb):
    grid = (N_NODES // BLK,)
    return pl.pallas_call(
        _tc_body,
        grid=grid,
        in_specs=[
            pl.BlockSpec((BLK, x.shape[1]), lambda i: (i, 0)),
            pl.BlockSpec(Wt.shape, lambda i: (0, 0)),
            pl.BlockSpec(b2.shape, lambda i: (0, 0)),
            pl.BlockSpec((BLK, 1), lambda i: (i, 0)),
            pl.BlockSpec((BLK, 1), lambda i: (i, 0)),
            pl.BlockSpec(in_emb.shape, lambda i: (0, 0)),
            pl.BlockSpec(out_emb.shape, lambda i: (0, 0)),
        ],
        out_specs=pl.BlockSpec((BLK, EMB_DIM), lambda i: (i, 0)),
        out_shape=jax.ShapeDtypeStruct((N_NODES, EMB_DIM), jnp.float32),
    )(x, Wt, b2, in_deg, out_deg, in_emb, out_emb)


def kernel(x, edge_index, W, b, in_emb, out_emb):
    src = edge_index[0].astype(jnp.int32)
    dst = edge_index[1].astype(jnp.int32)
    not_self = (src != dst).astype(jnp.int32)
    in_deg = jnp.zeros((N_NODES,), dtype=jnp.int32).at[dst].add(not_self)
    out_deg = jnp.zeros((N_NODES,), dtype=jnp.int32).at[src].add(not_self)
    in_deg = jnp.clip(in_deg, 0, MAX_DEG - 1).reshape(N_NODES, 1)
    out_deg = jnp.clip(out_deg, 0, MAX_DEG - 1).reshape(N_NODES, 1)
    return _tc_pass(x, W.T, b.reshape(1, EMB_DIM), in_deg, out_deg, in_emb, out_emb)


# speedup vs baseline: 1.9877x; 1.1149x over previous
import functools

import jax
import jax.numpy as jnp
from jax import lax
from jax.experimental import pallas as pl
from jax.experimental.pallas import tpu as pltpu
from jax.experimental.pallas import tpu_sc as plsc

N_NODES = 100000
N_PAD = 102400
EMB_DIM = 128
MAX_DEG = 256
BLK = 2000

E_PAD = 1638400
CHUNK = 2048
E_PER_SUB = E_PAD // 16
NCHUNKS = E_PER_SUB // CHUNK
SEG = N_PAD // 16


_sc_mesh = plsc.VectorSubcoreMesh(core_axis_name="c", subcore_axis_name="s")


@functools.partial(
    pl.kernel,
    out_type=jax.ShapeDtypeStruct((2, N_PAD), jnp.int32),
    mesh=_sc_mesh,
    scratch_types=[
        pltpu.VMEM_SHARED((N_PAD,), jnp.int32),
        pltpu.VMEM((CHUNK,), jnp.int32),
        pltpu.VMEM((CHUNK,), jnp.int32),
        pltpu.VMEM((CHUNK,), jnp.int32),
    ],
)
def _sc_degrees(srcH, dstH, zerosH, outH, acc, src_v, dst_v, val_v):
    c = lax.axis_index("c")
    s = lax.axis_index("s")

    pltpu.sync_copy(zerosH.at[pl.ds(s * SEG, SEG)], acc.at[pl.ds(s * SEG, SEG)])
    plsc.subcore_barrier()

    def chunk_body(t, carry):
        e0 = s * E_PER_SUB + t * CHUNK
        pltpu.sync_copy(srcH.at[pl.ds(e0, CHUNK)], src_v)
        pltpu.sync_copy(dstH.at[pl.ds(e0, CHUNK)], dst_v)

        def vec_body(l, carry2):
            sv = src_v[pl.ds(l * 16, 16)]
            dv = dst_v[pl.ds(l * 16, 16)]
            val_v[pl.ds(l * 16, 16)] = jnp.where(
                sv != dv, jnp.int32(1), jnp.int32(0))
            return carry2

        lax.fori_loop(0, CHUNK // 16, vec_body, 0)

        @pl.when(c == 0)
        def _():
            pltpu.sync_copy(val_v, acc.at[dst_v], add=True)

        @pl.when(c == 1)
        def _():
            pltpu.sync_copy(val_v, acc.at[src_v], add=True)

        return carry

    lax.fori_loop(0, NCHUNKS, chunk_body, 0)
    plsc.subcore_barrier()
    pltpu.sync_copy(acc.at[pl.ds(s * SEG, SEG)], outH.at[c, pl.ds(s * SEG, SEG)])


def _tc_body(x_ref, wt_ref, b_ref, ind_ref, outd_ref, ie_ref, oe_ref, o_ref):
    h = jnp.dot(x_ref[...], wt_ref[...], preferred_element_type=jnp.float32)
    h = h + b_ref[...]
    iota = lax.broadcasted_iota(jnp.int32, (BLK, MAX_DEG), 1)
    ind = jnp.clip(ind_ref[...], 0, MAX_DEG - 1)
    outd = jnp.clip(outd_ref[...], 0, MAX_DEG - 1)
    oh_in = (ind == iota).astype(jnp.float32)
    oh_out = (outd == iota).astype(jnp.float32)
    h = h + jnp.dot(oh_in, ie_ref[...], preferred_element_type=jnp.float32)
    h = h + jnp.dot(oh_out, oe_ref[...], preferred_element_type=jnp.float32)
    o_ref[...] = h


def _tc_pass(x, Wt, b2, in_deg, out_deg, in_emb, out_emb):
    grid = (N_NODES // BLK,)
    return pl.pallas_call(
        _tc_body,
        grid=grid,
        in_specs=[
            pl.BlockSpec((BLK, x.shape[1]), lambda i: (i, 0)),
            pl.BlockSpec(Wt.shape, lambda i: (0, 0)),
            pl.BlockSpec(b2.shape, lambda i: (0, 0)),
            pl.BlockSpec((BLK, 1), lambda i: (i, 0)),
            pl.BlockSpec((BLK, 1), lambda i: (i, 0)),
            pl.BlockSpec(in_emb.shape, lambda i: (0, 0)),
            pl.BlockSpec(out_emb.shape, lambda i: (0, 0)),
        ],
        out_specs=pl.BlockSpec((BLK, EMB_DIM), lambda i: (i, 0)),
        out_shape=jax.ShapeDtypeStruct((N_NODES, EMB_DIM), jnp.float32),
    )(x, Wt, b2, in_deg, out_deg, in_emb, out_emb)


def kernel(x, edge_index, W, b, in_emb, out_emb):
    n_edges = edge_index.shape[1]
    pad = E_PAD - n_edges
    src = jnp.pad(edge_index[0].astype(jnp.int32), (0, pad))
    dst = jnp.pad(edge_index[1].astype(jnp.int32), (0, pad))
    zeros = jnp.zeros((N_PAD,), jnp.int32)
    deg = _sc_degrees(src, dst, zeros)
    in_deg = deg[0, :N_NODES].reshape(N_NODES, 1)
    out_deg = deg[1, :N_NODES].reshape(N_NODES, 1)
    return _tc_pass(x, W.T, b.reshape(1, EMB_DIM), in_deg, out_deg, in_emb, out_emb)


# speedup vs baseline: 3.1636x; 1.5916x over previous
import functools

import jax
import jax.numpy as jnp
from jax import lax
from jax.experimental import pallas as pl
from jax.experimental.pallas import tpu as pltpu
from jax.experimental.pallas import tpu_sc as plsc

N_NODES = 100000
N_PAD = 102400
EMB_DIM = 128
MAX_DEG = 256
BLK = 2048

E_TOTAL = 1600000
CHUNK = 2000
E_PER_SUB = E_TOTAL // 16
NCHUNKS = E_PER_SUB // CHUNK
SEG = N_PAD // 16


_sc_mesh = plsc.VectorSubcoreMesh(core_axis_name="c", subcore_axis_name="s")


@functools.partial(
    pl.kernel,
    out_type=jax.ShapeDtypeStruct((2, N_PAD), jnp.int32),
    mesh=_sc_mesh,
    scratch_types=[
        pltpu.VMEM_SHARED((N_PAD,), jnp.int32),
        pltpu.VMEM((CHUNK,), jnp.int32),
        pltpu.VMEM((CHUNK,), jnp.int32),
        pltpu.VMEM((CHUNK,), jnp.int32),
    ],
)
def _sc_degrees(edgesH, zerosH, outH, acc, src_v, dst_v, val_v):
    c = lax.axis_index("c")
    s = lax.axis_index("s")

    pltpu.sync_copy(zerosH.at[pl.ds(s * SEG, SEG)], acc.at[pl.ds(s * SEG, SEG)])
    plsc.subcore_barrier()

    def chunk_body(t, carry):
        e0 = s * E_PER_SUB + t * CHUNK
        pltpu.sync_copy(edgesH.at[pl.ds(e0, CHUNK)], src_v)
        pltpu.sync_copy(edgesH.at[pl.ds(E_TOTAL + e0, CHUNK)], dst_v)

        def vec_body(i, carry2):
            for u in range(5):
                o = i * 80 + u * 16
                sv = src_v[pl.ds(o, 16)]
                dv = dst_v[pl.ds(o, 16)]
                val_v[pl.ds(o, 16)] = jnp.where(
                    sv != dv, jnp.int32(1), jnp.int32(0))
            return carry2

        lax.fori_loop(0, CHUNK // 80, vec_body, 0)

        @pl.when(c == 0)
        def _():
            pltpu.sync_copy(val_v, acc.at[dst_v], add=True)

        @pl.when(c == 1)
        def _():
            pltpu.sync_copy(val_v, acc.at[src_v], add=True)

        return carry

    lax.fori_loop(0, NCHUNKS, chunk_body, 0)
    plsc.subcore_barrier()
    pltpu.sync_copy(acc.at[pl.ds(s * SEG, SEG)], outH.at[c, pl.ds(s * SEG, SEG)])


def _tc_body(x_ref, wt_ref, b_ref, ind_ref, outd_ref, ie_ref, oe_ref, o_ref):
    h = jnp.dot(x_ref[...], wt_ref[...], preferred_element_type=jnp.float32)
    h = h + b_ref[...]
    iota = lax.broadcasted_iota(jnp.int32, (MAX_DEG, BLK), 0)
    ind = jnp.clip(ind_ref[0], 0, MAX_DEG - 1)
    outd = jnp.clip(outd_ref[0], 0, MAX_DEG - 1)
    dn = (((0,), (0,)), ((), ()))
    oh_in = (ind == iota).astype(jnp.bfloat16)
    oh_out = (outd == iota).astype(jnp.bfloat16)
    h = h + lax.dot_general(oh_in, ie_ref[...], dn,
                            preferred_element_type=jnp.float32)
    h = h + lax.dot_general(oh_out, oe_ref[...], dn,
                            preferred_element_type=jnp.float32)
    o_ref[...] = h


def _tc_pass(x, Wt, b2, deg, in_emb, out_emb):
    grid = (pl.cdiv(N_NODES, BLK),)
    return pl.pallas_call(
        _tc_body,
        grid=grid,
        in_specs=[
            pl.BlockSpec((BLK, x.shape[1]), lambda i: (i, 0)),
            pl.BlockSpec(Wt.shape, lambda i: (0, 0)),
            pl.BlockSpec(b2.shape, lambda i: (0, 0)),
            pl.BlockSpec((1, 1, BLK), lambda i: (0, 0, i)),
            pl.BlockSpec((1, 1, BLK), lambda i: (1, 0, i)),
            pl.BlockSpec(in_emb.shape, lambda i: (0, 0)),
            pl.BlockSpec(out_emb.shape, lambda i: (0, 0)),
        ],
        out_specs=pl.BlockSpec((BLK, EMB_DIM), lambda i: (i, 0)),
        out_shape=jax.ShapeDtypeStruct((N_NODES, EMB_DIM), jnp.float32),
    )(x, Wt, b2, deg, deg, in_emb, out_emb)


def kernel(x, edge_index, W, b, in_emb, out_emb):
    edges = edge_index.astype(jnp.int32).reshape(-1)
    zeros = jnp.zeros((N_PAD,), jnp.int32)
    deg = _sc_degrees(edges, zeros).reshape(2, 1, N_PAD)
    return _tc_pass(x, W.T, b.reshape(1, EMB_DIM), deg,
                    in_emb.astype(jnp.bfloat16), out_emb.astype(jnp.bfloat16))


# speedup vs baseline: 3.6663x; 1.1589x over previous
import functools

import jax
import jax.numpy as jnp
from jax import lax
from jax.experimental import pallas as pl
from jax.experimental.pallas import tpu as pltpu
from jax.experimental.pallas import tpu_sc as plsc

N_NODES = 100000
N_PAD = 102400
EMB_DIM = 128
MAX_DEG = 256
BLK = 2048

E_TOTAL = 1600000
CHUNK = 2000
E_PER_SUB = E_TOTAL // 16
NCHUNKS = E_PER_SUB // CHUNK
NPAIRS = NCHUNKS // 2
SEG = N_PAD // 16


_sc_mesh = plsc.VectorSubcoreMesh(core_axis_name="c", subcore_axis_name="s")


@functools.partial(
    pl.kernel,
    out_type=jax.ShapeDtypeStruct((2, N_PAD), jnp.int32),
    mesh=_sc_mesh,
    scratch_types=[
        pltpu.VMEM_SHARED((N_PAD,), jnp.int32),
        pltpu.VMEM((CHUNK,), jnp.int32),
        pltpu.VMEM((CHUNK,), jnp.int32),
        pltpu.VMEM((CHUNK,), jnp.int32),
        pltpu.VMEM((CHUNK,), jnp.int32),
        pltpu.VMEM((CHUNK,), jnp.int32),
        pltpu.VMEM((CHUNK,), jnp.int32),
        pltpu.SemaphoreType.DMA,
        pltpu.SemaphoreType.DMA,
    ],
)
def _sc_degrees(edgesH, outH, acc,
                src0, dst0, val0, src1, dst1, val1, sem0, sem1):
    c = lax.axis_index("c")
    s = lax.axis_index("s")
    bufs = ((src0, dst0, val0, sem0), (src1, dst1, val1, sem1))

    def load_pair(chunk, sbuf, dbuf, sem):
        e0 = s * E_PER_SUB + chunk * CHUNK
        pltpu.async_copy(edgesH.at[pl.ds(e0, CHUNK)], sbuf, sem)
        pltpu.async_copy(edgesH.at[pl.ds(E_TOTAL + e0, CHUNK)], dbuf, sem)

    def wait_pair(sbuf, dbuf, sem):
        pltpu.make_async_copy(edgesH.at[pl.ds(0, CHUNK)], sbuf, sem).wait()
        pltpu.make_async_copy(edgesH.at[pl.ds(0, CHUNK)], dbuf, sem).wait()

    load_pair(0, src0, dst0, sem0)
    load_pair(1, src1, dst1, sem1)

    def zero_body(i, carry):
        val0[pl.ds(i * 16, 16)] = jnp.zeros((16,), jnp.int32)
        return carry

    lax.fori_loop(0, CHUNK // 16, zero_body, 0)
    base = s * SEG
    for k in range(3):
        pltpu.sync_copy(val0, acc.at[pl.ds(base + k * CHUNK, CHUNK)])
    pltpu.sync_copy(val0.at[pl.ds(0, 400)], acc.at[pl.ds(base + 3 * CHUNK, 400)])
    plsc.subcore_barrier()

    def pair_body(t, carry):
        for b in range(2):
            sbuf, dbuf, vbuf, sem = bufs[b]
            chunk = t * 2 + b
            wait_pair(sbuf, dbuf, sem)

            def vec_body(i, carry2):
                for u in range(5):
                    o = i * 80 + u * 16
                    sv = sbuf[pl.ds(o, 16)]
                    dv = dbuf[pl.ds(o, 16)]
                    vbuf[pl.ds(o, 16)] = jnp.where(
                        sv != dv, jnp.int32(1), jnp.int32(0))
                return carry2

            lax.fori_loop(0, CHUNK // 80, vec_body, 0)

            @pl.when(c == 0)
            def _():
                pltpu.sync_copy(vbuf, acc.at[dbuf], add=True)

            @pl.when(c == 1)
            def _():
                pltpu.sync_copy(vbuf, acc.at[sbuf], add=True)

            @pl.when(chunk + 2 < NCHUNKS)
            def _():
                load_pair(chunk + 2, sbuf, dbuf, sem)
        return carry

    lax.fori_loop(0, NPAIRS, pair_body, 0)
    plsc.subcore_barrier()
    pltpu.sync_copy(acc.at[pl.ds(base, SEG)], outH.at[c, pl.ds(base, SEG)])


def _proj_body(x_ref, wt_ref, b_ref, o_ref):
    o_ref[...] = jnp.dot(x_ref[...], wt_ref[...],
                         preferred_element_type=jnp.float32) + b_ref[...]


def _tc_proj(x, Wt, b2):
    grid = (pl.cdiv(N_NODES, BLK),)
    return pl.pallas_call(
        _proj_body,
        grid=grid,
        in_specs=[
            pl.BlockSpec((BLK, x.shape[1]), lambda i: (i, 0)),
            pl.BlockSpec(Wt.shape, lambda i: (0, 0)),
            pl.BlockSpec(b2.shape, lambda i: (0, 0)),
        ],
        out_specs=pl.BlockSpec((BLK, EMB_DIM), lambda i: (i, 0)),
        out_shape=jax.ShapeDtypeStruct((N_NODES, EMB_DIM), jnp.float32),
    )(x, Wt, b2)


def _lookup_body(h_ref, ind_ref, outd_ref, ie_ref, oe_ref, o_ref):
    iota = lax.broadcasted_iota(jnp.int32, (MAX_DEG, BLK), 0)
    ind = jnp.clip(ind_ref[0], 0, MAX_DEG - 1)
    outd = jnp.clip(outd_ref[0], 0, MAX_DEG - 1)
    dn = (((0,), (0,)), ((), ()))
    oh_in = (ind == iota).astype(jnp.bfloat16)
    oh_out = (outd == iota).astype(jnp.bfloat16)
    h = h_ref[...]
    h = h + lax.dot_general(oh_in, ie_ref[...], dn,
                            preferred_element_type=jnp.float32)
    h = h + lax.dot_general(oh_out, oe_ref[...], dn,
                            preferred_element_type=jnp.float32)
    o_ref[...] = h


def _tc_lookup(h0, deg, in_emb, out_emb):
    grid = (pl.cdiv(N_NODES, BLK),)
    return pl.pallas_call(
        _lookup_body,
        grid=grid,
        in_specs=[
            pl.BlockSpec((BLK, EMB_DIM), lambda i: (i, 0)),
            pl.BlockSpec((1, 1, BLK), lambda i: (0, 0, i)),
            pl.BlockSpec((1, 1, BLK), lambda i: (1, 0, i)),
            pl.BlockSpec(in_emb.shape, lambda i: (0, 0)),
            pl.BlockSpec(out_emb.shape, lambda i: (0, 0)),
        ],
        out_specs=pl.BlockSpec((BLK, EMB_DIM), lambda i: (i, 0)),
        out_shape=jax.ShapeDtypeStruct((N_NODES, EMB_DIM), jnp.float32),
    )(h0, deg, deg, in_emb, out_emb)


def kernel(x, edge_index, W, b, in_emb, out_emb):
    edges = edge_index.astype(jnp.int32).reshape(-1)
    deg = _sc_degrees(edges).reshape(2, 1, N_PAD)
    h0 = _tc_proj(x, W.T, b.reshape(1, EMB_DIM))
    return _tc_lookup(h0, deg,
                      in_emb.astype(jnp.bfloat16), out_emb.astype(jnp.bfloat16))
